# baseline (speedup 1.0000x reference)
import functools

import jax
import jax.numpy as jnp
from jax import lax
from jax.experimental import pallas as pl
from jax.experimental.pallas import tpu as pltpu
from jax.experimental.pallas import tpu_sc as plsc

VOCAB = 16384
D_MODEL = 512
NUM_CORES = 2
NUM_SUBCORES = 16
NW = NUM_CORES * NUM_SUBCORES


_BLK = 4096


def _combine_body(mask_ref, gw_ref, res_ref, gate_ref, out_ref):
    geom = jnp.dot(mask_ref[...], gw_ref[...],
                   preferred_element_type=jnp.float32)
    g = jax.nn.sigmoid(gate_ref[...])
    out_ref[...] = res_ref[...] + g * geom


def _build_combined(mask_table, geom_W, residual, gate):
    return pl.pallas_call(
        _combine_body,
        grid=(VOCAB // _BLK,),
        in_specs=[
            pl.BlockSpec((_BLK, 8), lambda i: (i, 0)),
            pl.BlockSpec((8, D_MODEL), lambda i: (0, 0)),
            pl.BlockSpec((_BLK, D_MODEL), lambda i: (i, 0)),
            pl.BlockSpec((_BLK, 1), lambda i: (i, 0)),
        ],
        out_specs=pl.BlockSpec((_BLK, D_MODEL), lambda i: (i, 0)),
        out_shape=jax.ShapeDtypeStruct((VOCAB, D_MODEL), jnp.float32),
    )(mask_table, geom_W, residual, gate)


_CHUNK = 64
_NBUF = 3


def _make_gather(total_tokens):
    b_per_w = total_tokens // NW
    nchunk = b_per_w // _CHUNK
    mesh = plsc.VectorSubcoreMesh(core_axis_name="c", subcore_axis_name="s")

    @functools.partial(
        pl.kernel,
        out_type=jax.ShapeDtypeStruct((total_tokens, D_MODEL), jnp.float32),
        mesh=mesh,
        scratch_types=[
            pltpu.VMEM((nchunk, _CHUNK), jnp.int32),
        ] + [pltpu.VMEM((_CHUNK, D_MODEL), jnp.float32)] * _NBUF
          + [pltpu.SemaphoreType.DMA] * (2 * _NBUF),
    )
    def _gather(table_hbm, idx_hbm, out_hbm, idx_v, *bufs_sems):
        bufs = bufs_sems[:_NBUF]
        gsems = bufs_sems[_NBUF:2 * _NBUF]
        wsems = bufs_sems[2 * _NBUF:]
        wid = lax.axis_index("s") * NUM_CORES + lax.axis_index("c")
        base = wid * b_per_w
        pltpu.sync_copy(idx_hbm.at[wid], idx_v)

        def out_slice(jj):
            return out_hbm.at[pl.ds(base + jj * _CHUNK, _CHUNK)]

        for b in range(_NBUF - 1):
            pltpu.async_copy(table_hbm.at[idx_v.at[b]], bufs[b], gsems[b])

        @pl.loop(0, nchunk, step=_NBUF)
        def _(j):
            for b in range(_NBUF):
                jj = j + b
                nxt = jj + _NBUF - 1
                pb = (b + _NBUF - 1) % _NBUF

                @pl.when(nxt < nchunk)
                def _():
                    @pl.when(nxt >= _NBUF)
                    def _():
                        pltpu.make_async_copy(
                            bufs[pb], out_slice(nxt - _NBUF), wsems[pb]).wait()
                    pltpu.async_copy(
                        table_hbm.at[idx_v.at[nxt]], bufs[pb], gsems[pb])

                @pl.when(jj < nchunk)
                def _():
                    pltpu.make_async_copy(
                        table_hbm.at[idx_v.at[jj]], bufs[b], gsems[b]).wait()
                    pltpu.async_copy(bufs[b], out_slice(jj), wsems[b])

        for jj in range(nchunk - _NBUF, nchunk):
            b = jj % _NBUF
            pltpu.make_async_copy(bufs[b], out_slice(jj), wsems[b]).wait()

    return _gather


def kernel(token_ids, mask_table, geom_W, residual, gate):
    batch, seq = token_ids.shape
    total = batch * seq
    combined = _build_combined(mask_table, geom_W, residual, gate)
    idx = token_ids.reshape(NW, total // NW // _CHUNK, _CHUNK)
    out = _make_gather(total)(combined, idx)
    return out.reshape(batch, seq, D_MODEL)
